# baseline (speedup 1.0000x reference)
import functools
import math

import jax
import jax.numpy as jnp
from jax.experimental import pallas as pl
from jax.experimental.pallas import tpu as pltpu

_TWO_PI = 2.0 * math.pi
_SIGMA_DATA = 1.0


def _upsample_body(sig_ref, lr_ref, out_ref, *, width):
    b = pl.program_id(0)
    s = sig_ref[b]
    c_in = jax.lax.rsqrt(s * s + _SIGMA_DATA * _SIGMA_DATA)

    wi = jax.lax.broadcasted_iota(jnp.int32, (width, 2 * width), 0)
    ci = jax.lax.broadcasted_iota(jnp.int32, (width, 2 * width), 1)
    d = (ci // 2 == wi).astype(jnp.bfloat16)
    y = jnp.dot(lr_ref[...].astype(jnp.bfloat16), d,
                preferred_element_type=jnp.float32) * c_in
    out_ref[...] = jnp.repeat(y, 2, axis=0).astype(out_ref.dtype)


def _mapping_body(sig_ref, w_ref, pool_ref, out_ref, *, half):
    c_noise = jnp.log1p(sig_ref[...])
    f = _TWO_PI * (c_noise * w_ref[...].astype(jnp.float32))
    out_ref[...] = jnp.concatenate(
        [jnp.cos(f), jnp.sin(f), pool_ref[...].astype(jnp.float32)], axis=-1)


def kernel(input, sigma, low_res, low_res_sigma, cross_cond,
           cross_cond_padding, pooler, fourier_weight):
    B, C, H, W = low_res.shape
    G = C * H
    half = fourier_weight.shape[0]
    P = pooler.shape[-1]
    out_dtype = low_res.dtype

    lr = low_res.reshape(B, G, W)
    sig32 = low_res_sigma.astype(jnp.float32)

    up = pl.pallas_call(
        functools.partial(_upsample_body, width=W),
        out_shape=jax.ShapeDtypeStruct((B, 2 * G, 2 * W), out_dtype),
        grid_spec=pltpu.PrefetchScalarGridSpec(
            num_scalar_prefetch=1,
            grid=(B,),
            in_specs=[pl.BlockSpec((None, G, W), lambda b, sig: (b, 0, 0))],
            out_specs=pl.BlockSpec((None, 2 * G, 2 * W),
                                   lambda b, sig: (b, 0, 0)),
        ),
        compiler_params=pltpu.CompilerParams(
            dimension_semantics=("parallel",),
            vmem_limit_bytes=32 * 1024 * 1024,
        ),
    )(sig32, lr)

    mapping_cond = pl.pallas_call(
        functools.partial(_mapping_body, half=half),
        out_shape=jax.ShapeDtypeStruct((B, 2 * half + P), jnp.float32),
        in_specs=[pl.BlockSpec(memory_space=pltpu.MemorySpace.VMEM)] * 3,
        out_specs=pl.BlockSpec(memory_space=pltpu.MemorySpace.VMEM),
    )(sig32[:, None], fourier_weight.reshape(1, half), pooler)

    return {
        "input": input,
        "sigma": sigma,
        "unet_cond": up.reshape(B, C, 2 * H, 2 * W),
        "mapping_cond": mapping_cond,
        "cross_cond": cross_cond,
        "cross_cond_padding": cross_cond_padding,
    }


# speedup vs baseline: 1.1494x; 1.1494x over previous
import functools
import math

import jax
import jax.numpy as jnp
from jax.experimental import pallas as pl
from jax.experimental.pallas import tpu as pltpu

_TWO_PI = 2.0 * math.pi
_SIGMA_DATA = 1.0


def _upsample_body(sig_ref, lr_ref, out_ref, *, width):
    b = pl.program_id(0)
    s = sig_ref[b]
    c_in = jax.lax.rsqrt(s * s + _SIGMA_DATA * _SIGMA_DATA)

    xs = lr_ref[...] * c_in
    x2 = pltpu.bitcast(
        pltpu.pack_elementwise([xs, xs], packed_dtype=jnp.bfloat16),
        jnp.bfloat16)

    wi = jax.lax.broadcasted_iota(jnp.int32, (width, 2 * width), 0)
    ci = jax.lax.broadcasted_iota(jnp.int32, (width, 2 * width), 1)
    d = (ci // 2 == wi).astype(jnp.bfloat16)
    out_ref[...] = jnp.dot(x2, d, preferred_element_type=jnp.float32
                           ).astype(out_ref.dtype)


def _mapping_body(sig_ref, w_ref, pool_ref, out_ref, *, half):
    c_noise = jnp.log1p(sig_ref[...])
    f = _TWO_PI * (c_noise * w_ref[...].astype(jnp.float32))
    out_ref[...] = jnp.concatenate(
        [jnp.cos(f), jnp.sin(f), pool_ref[...].astype(jnp.float32)], axis=-1)


def kernel(input, sigma, low_res, low_res_sigma, cross_cond,
           cross_cond_padding, pooler, fourier_weight):
    B, C, H, W = low_res.shape
    G = C * H
    half = fourier_weight.shape[0]
    P = pooler.shape[-1]
    out_dtype = low_res.dtype

    lr = low_res.reshape(B, G, W)
    sig32 = low_res_sigma.astype(jnp.float32)

    up = pl.pallas_call(
        functools.partial(_upsample_body, width=W),
        out_shape=jax.ShapeDtypeStruct((B, 2 * G, 2 * W), out_dtype),
        grid_spec=pltpu.PrefetchScalarGridSpec(
            num_scalar_prefetch=1,
            grid=(B,),
            in_specs=[pl.BlockSpec((None, G, W), lambda b, sig: (b, 0, 0))],
            out_specs=pl.BlockSpec((None, 2 * G, 2 * W),
                                   lambda b, sig: (b, 0, 0)),
        ),
        compiler_params=pltpu.CompilerParams(
            dimension_semantics=("parallel",),
            vmem_limit_bytes=32 * 1024 * 1024,
        ),
    )(sig32, lr)

    mapping_cond = pl.pallas_call(
        functools.partial(_mapping_body, half=half),
        out_shape=jax.ShapeDtypeStruct((B, 2 * half + P), jnp.float32),
        in_specs=[pl.BlockSpec(memory_space=pltpu.MemorySpace.VMEM)] * 3,
        out_specs=pl.BlockSpec(memory_space=pltpu.MemorySpace.VMEM),
    )(sig32[:, None], fourier_weight.reshape(1, half), pooler)

    return {
        "input": input,
        "sigma": sigma,
        "unet_cond": up.reshape(B, C, 2 * H, 2 * W),
        "mapping_cond": mapping_cond,
        "cross_cond": cross_cond,
        "cross_cond_padding": cross_cond_padding,
    }


# speedup vs baseline: 1.3552x; 1.1791x over previous
import functools
import math

import jax
import jax.numpy as jnp
from jax.experimental import pallas as pl
from jax.experimental.pallas import tpu as pltpu

_TWO_PI = 2.0 * math.pi
_SIGMA_DATA = 1.0


def _upsample_body(sig_ref, lr_ref, out_ref, *, width, bpb):
    b = pl.program_id(0)

    wi = jax.lax.broadcasted_iota(jnp.int32, (width, 2 * width), 0)
    ci = jax.lax.broadcasted_iota(jnp.int32, (width, 2 * width), 1)
    d = (ci // 2 == wi).astype(jnp.bfloat16)

    for i in range(bpb):
        s = sig_ref[b * bpb + i]
        c_in = jax.lax.rsqrt(s * s + _SIGMA_DATA * _SIGMA_DATA)
        xs = lr_ref[i] * c_in
        x2 = pltpu.bitcast(
            pltpu.pack_elementwise([xs, xs], packed_dtype=jnp.bfloat16),
            jnp.bfloat16)
        out_ref[i] = jnp.dot(x2, d, preferred_element_type=jnp.float32
                             ).astype(out_ref.dtype)


def _mapping_body(sig_ref, w_ref, pool_ref, out_ref, *, half):
    c_noise = jnp.log1p(sig_ref[...])
    f = _TWO_PI * (c_noise * w_ref[...].astype(jnp.float32))
    out_ref[...] = jnp.concatenate(
        [jnp.cos(f), jnp.sin(f), pool_ref[...].astype(jnp.float32)], axis=-1)


def kernel(input, sigma, low_res, low_res_sigma, cross_cond,
           cross_cond_padding, pooler, fourier_weight):
    B, C, H, W = low_res.shape
    G = C * H
    half = fourier_weight.shape[0]
    P = pooler.shape[-1]
    out_dtype = low_res.dtype

    lr = low_res.reshape(B, G, W)
    sig32 = low_res_sigma.astype(jnp.float32)

    BPB = 4
    up = pl.pallas_call(
        functools.partial(_upsample_body, width=W, bpb=BPB),
        out_shape=jax.ShapeDtypeStruct((B, 2 * G, 2 * W), out_dtype),
        grid_spec=pltpu.PrefetchScalarGridSpec(
            num_scalar_prefetch=1,
            grid=(B // BPB,),
            in_specs=[pl.BlockSpec((BPB, G, W), lambda b, sig: (b, 0, 0))],
            out_specs=pl.BlockSpec((BPB, 2 * G, 2 * W),
                                   lambda b, sig: (b, 0, 0)),
        ),
        compiler_params=pltpu.CompilerParams(
            dimension_semantics=("parallel",),
            vmem_limit_bytes=32 * 1024 * 1024,
        ),
    )(sig32, lr)

    mapping_cond = pl.pallas_call(
        functools.partial(_mapping_body, half=half),
        out_shape=jax.ShapeDtypeStruct((B, 2 * half + P), jnp.float32),
        in_specs=[pl.BlockSpec(memory_space=pltpu.MemorySpace.VMEM)] * 3,
        out_specs=pl.BlockSpec(memory_space=pltpu.MemorySpace.VMEM),
    )(sig32[:, None], fourier_weight.reshape(1, half), pooler)

    return {
        "input": input,
        "sigma": sigma,
        "unet_cond": up.reshape(B, C, 2 * H, 2 * W),
        "mapping_cond": mapping_cond,
        "cross_cond": cross_cond,
        "cross_cond_padding": cross_cond_padding,
    }


# speedup vs baseline: 1.3556x; 1.0003x over previous
import functools
import math

import jax
import jax.numpy as jnp
from jax.experimental import pallas as pl
from jax.experimental.pallas import tpu as pltpu

_TWO_PI = 2.0 * math.pi
_SIGMA_DATA = 1.0


def _upsample_body(sig_ref, lr_ref, out_ref, *, width, bpb):
    b = pl.program_id(0)

    wi = jax.lax.broadcasted_iota(jnp.int32, (width, 2 * width), 0)
    ci = jax.lax.broadcasted_iota(jnp.int32, (width, 2 * width), 1)
    d = (ci // 2 == wi).astype(jnp.bfloat16)

    for i in range(bpb):
        s = sig_ref[b * bpb + i]
        c_in = jax.lax.rsqrt(s * s + _SIGMA_DATA * _SIGMA_DATA)
        xs = lr_ref[i] * c_in
        x2 = pltpu.bitcast(
            pltpu.pack_elementwise([xs, xs], packed_dtype=jnp.bfloat16),
            jnp.bfloat16)
        out_ref[i] = jnp.dot(x2, d, preferred_element_type=jnp.float32
                             ).astype(out_ref.dtype)


def _mapping_body(sig_ref, w_ref, pool_ref, out_ref, *, half):
    c_noise = jnp.log1p(sig_ref[...])
    f = _TWO_PI * (c_noise * w_ref[...].astype(jnp.float32))
    out_ref[...] = jnp.concatenate(
        [jnp.cos(f), jnp.sin(f), pool_ref[...].astype(jnp.float32)], axis=-1)


def kernel(input, sigma, low_res, low_res_sigma, cross_cond,
           cross_cond_padding, pooler, fourier_weight):
    B, C, H, W = low_res.shape
    G = C * H
    half = fourier_weight.shape[0]
    P = pooler.shape[-1]
    out_dtype = low_res.dtype

    lr = low_res.reshape(B, G, W)
    sig32 = low_res_sigma.astype(jnp.float32)

    BPB = 8
    up = pl.pallas_call(
        functools.partial(_upsample_body, width=W, bpb=BPB),
        out_shape=jax.ShapeDtypeStruct((B, 2 * G, 2 * W), out_dtype),
        grid_spec=pltpu.PrefetchScalarGridSpec(
            num_scalar_prefetch=1,
            grid=(B // BPB,),
            in_specs=[pl.BlockSpec((BPB, G, W), lambda b, sig: (b, 0, 0))],
            out_specs=pl.BlockSpec((BPB, 2 * G, 2 * W),
                                   lambda b, sig: (b, 0, 0)),
        ),
        compiler_params=pltpu.CompilerParams(
            dimension_semantics=("parallel",),
            vmem_limit_bytes=32 * 1024 * 1024,
        ),
    )(sig32, lr)

    mapping_cond = pl.pallas_call(
        functools.partial(_mapping_body, half=half),
        out_shape=jax.ShapeDtypeStruct((B, 2 * half + P), jnp.float32),
        in_specs=[pl.BlockSpec(memory_space=pltpu.MemorySpace.VMEM)] * 3,
        out_specs=pl.BlockSpec(memory_space=pltpu.MemorySpace.VMEM),
    )(sig32[:, None], fourier_weight.reshape(1, half), pooler)

    return {
        "input": input,
        "sigma": sigma,
        "unet_cond": up.reshape(B, C, 2 * H, 2 * W),
        "mapping_cond": mapping_cond,
        "cross_cond": cross_cond,
        "cross_cond_padding": cross_cond_padding,
    }
